# Initial kernel scaffold; baseline (speedup 1.0000x reference)
import jax
import jax.numpy as jnp
from jax.experimental import pallas as pl


def kernel(x, edge_index, gamma, beta, W_in, b_in, W_out, b_out):
    raise NotImplementedError("write your pallas kernel here")


# speedup vs baseline: 134.6621x; 134.6621x over previous
import jax
import jax.numpy as jnp
from jax import lax
from jax.experimental import pallas as pl
from jax.experimental.pallas import tpu as pltpu
from jax.experimental.pallas import tpu_sc as plsc

NC = 2
NS = 16
_CMSG = 2000


def _msg_body(u_hbm, eidx, acc_out, gidx_v, sidx_v, rows_v, sem,
              shared_u, shared_acc):
    d = lax.axis_index("c")
    s = lax.axis_index("s")
    N = u_hbm.shape[1]
    E = eidx.shape[0] // 2
    C = gidx_v.shape[0]

    chunk = ((N // NS) + 7) // 8 * 8
    last = N - chunk * (NS - 1)

    @pl.when(s < NS - 1)
    def _():
        r0 = pl.multiple_of(s * chunk, 8)
        pltpu.sync_copy(u_hbm.at[d, pl.ds(r0, chunk), :],
                        shared_u.at[pl.ds(r0, chunk), :])
        pltpu.sync_copy(u_hbm.at[d, pl.ds(r0, chunk), :],
                        shared_acc.at[pl.ds(r0, chunk), :])

    @pl.when(s == NS - 1)
    def _():
        r0 = chunk * (NS - 1)
        pltpu.sync_copy(u_hbm.at[d, pl.ds(r0, last), :],
                        shared_u.at[pl.ds(r0, last), :])
        pltpu.sync_copy(u_hbm.at[d, pl.ds(r0, last), :],
                        shared_acc.at[pl.ds(r0, last), :])

    plsc.subcore_barrier()

    per_sub = E // NS

    def body(i, carry):
        gbase = pl.multiple_of(d * E + s * per_sub + i * C, 8)
        sbase = pl.multiple_of((1 - d) * E + s * per_sub + i * C, 8)
        pltpu.sync_copy(eidx.at[pl.ds(gbase, C)], gidx_v)
        pltpu.sync_copy(eidx.at[pl.ds(sbase, C)], sidx_v)
        pltpu.async_copy(shared_u.at[gidx_v], rows_v, sem).wait()
        pltpu.sync_copy(rows_v, shared_acc.at[sidx_v], add=True)
        return carry

    lax.fori_loop(0, per_sub // C, body, 0)
    plsc.subcore_barrier()

    @pl.when(s < NS - 1)
    def _():
        r0 = pl.multiple_of(s * chunk, 8)
        pltpu.sync_copy(shared_acc.at[pl.ds(r0, chunk), :],
                        acc_out.at[d, pl.ds(r0, chunk), :])

    @pl.when(s == NS - 1)
    def _():
        r0 = chunk * (NS - 1)
        pltpu.sync_copy(shared_acc.at[pl.ds(r0, last), :],
                        acc_out.at[d, pl.ds(r0, last), :])


def _msg_call(N, K):
    mesh = plsc.VectorSubcoreMesh(core_axis_name="c", subcore_axis_name="s")
    return pl.kernel(
        _msg_body,
        out_type=jax.ShapeDtypeStruct((NC, N, K), jnp.float32),
        mesh=mesh,
        compiler_params=pltpu.CompilerParams(use_tc_tiling_on_sc=False),
        scratch_types=[
            pltpu.VMEM((_CMSG,), jnp.int32),
            pltpu.VMEM((_CMSG,), jnp.int32),
            pltpu.VMEM((_CMSG, K), jnp.float32),
            pltpu.SemaphoreType.DMA,
            pltpu.VMEM_SHARED((N, K), jnp.float32),
            pltpu.VMEM_SHARED((N, K), jnp.float32),
        ],
    )


def _dense_body(xt_ref, deg_ref, g_ref, bt_ref, wc_ref, ut_ref, dis_ref):
    K, Nn = xt_ref.shape
    xr = jnp.maximum(xt_ref[...], 0.0)
    mean = jnp.mean(xr, axis=1, keepdims=True)
    var = jnp.mean((xr - mean) ** 2, axis=1, keepdims=True)
    y = (xr - mean) * lax.rsqrt(var + 1e-5) * g_ref[...] + bt_ref[...]
    ut = lax.dot_general(wc_ref[...], y, (((0,), (0,)), ((), ())),
                         preferred_element_type=jnp.float32)
    dis = lax.rsqrt(deg_ref[...])
    scale = jnp.concatenate([jnp.broadcast_to(dis[0:1], (K, Nn)),
                             jnp.broadcast_to(dis[1:2], (K, Nn))], axis=0)
    ut_ref[...] = ut * scale
    dis_ref[...] = dis


def _dense_call(N, K):
    return pl.pallas_call(
        _dense_body,
        out_shape=[
            jax.ShapeDtypeStruct((2 * K, N), jnp.float32),
            jax.ShapeDtypeStruct((NC, N), jnp.float32),
        ],
    )


def _final_body(acc_ref, dis_ref, bc_ref, out_ref):
    K = acc_ref.shape[1]
    dis = dis_ref[...]
    in_t = dis[0:1] * acc_ref[0] + bc_ref[0:K]
    out_t = dis[1:2] * acc_ref[1] + bc_ref[K:2 * K]
    out_ref[...] = jnp.concatenate([in_t, out_t], axis=0)


def _final_call(N, K):
    return pl.pallas_call(
        _final_body,
        out_shape=jax.ShapeDtypeStruct((2 * K, N), jnp.float32),
    )


def kernel(x, edge_index, gamma, beta, W_in, b_in, W_out, b_out):
    N, K = x.shape
    E = edge_index.shape[1]
    assert N % 8 == 0 and E % (NS * _CMSG) == 0

    eflat = edge_index.reshape(NC * E)
    ones3 = jnp.ones((NC, N, 1), jnp.float32)
    deg = _msg_call(N, 1)(ones3, eflat).reshape(NC, N)
    xt = x.T
    Wc = jnp.concatenate([W_in, W_out], axis=1)
    ut, dis = _dense_call(N, K)(xt, deg, gamma.reshape(K, 1),
                                beta.reshape(K, 1), Wc)
    u = jnp.transpose(ut.reshape(NC, K, N), (0, 2, 1))
    acc = _msg_call(N, K)(u, eflat)
    acc_t = jnp.transpose(acc, (0, 2, 1))
    bc = jnp.concatenate([b_in, b_out]).reshape(2 * K, 1)
    out_t = _final_call(N, K)(acc_t, dis, bc)
    return out_t.T
